# baseline (speedup 1.0000x reference)
import functools

import jax
import jax.numpy as jnp
from jax import lax
from jax.experimental import pallas as pl
from jax.experimental.pallas import tpu as pltpu
from jax.experimental.pallas import tpu_sc as plsc

B, C, D, H, W = 2, 32, 16, 128, 128
N = D * H * W
P = B * N
K = 128
R = P // K
NC, NS = 2, 16
NW = NC * NS
RW = R // NW
LANES = 16
TRY = 4
TV = TRY * W
TCH = B * D * H // TRY
TCW = TCH // NW


def _prep_body(flow_ref, idx_ref, frac_ref):
    i = pl.program_id(0)
    b = i // D
    z = i % D
    fx = flow_ref[0, 0, 0]
    fy = flow_ref[0, 1, 0]
    fz = flow_ref[0, 2, 0]
    xs = lax.broadcasted_iota(jnp.int32, (H, W), 1).astype(jnp.float32) + fx
    ys = lax.broadcasted_iota(jnp.int32, (H, W), 0).astype(jnp.float32) + fy
    zs = z.astype(jnp.float32) + fz
    bx = jnp.clip(jnp.floor(xs), 0.0, float(W - 2))
    by = jnp.clip(jnp.floor(ys), 0.0, float(H - 2))
    bz = jnp.clip(jnp.floor(zs), 0.0, float(D - 2))
    gx = jnp.clip(xs - bx, 0.0, 1.0)
    gy = jnp.clip(ys - by, 0.0, 1.0)
    gz = jnp.clip(zs - bz, 0.0, 1.0)
    base = (b * N
            + (bz.astype(jnp.int32) * H + by.astype(jnp.int32)) * W
            + bx.astype(jnp.int32))
    idx_ref[0] = base
    idx_ref[1] = base + W
    idx_ref[2] = base + H * W
    idx_ref[3] = base + H * W + W
    frac_ref[0] = gx
    frac_ref[1] = gy
    frac_ref[2] = gz


_prep = pl.pallas_call(
    _prep_body,
    grid=(B * D,),
    in_specs=[pl.BlockSpec((1, 3, 1, H, W), lambda i: (i // D, 0, i % D, 0, 0))],
    out_specs=[pl.BlockSpec((4, H, W), lambda i: (0, i, 0)),
               pl.BlockSpec((3, H, W), lambda i: (0, i, 0))],
    out_shape=[jax.ShapeDtypeStruct((4, R, K), jnp.int32),
               jax.ShapeDtypeStruct((3, R, K), jnp.float32)],
)


def _lane(v, m):
    return lax.squeeze(lax.slice_in_dim(v, m, m + 1), (0,))


def _tr_body(in_hbm, table_hbm, vbuf, tbuf, isem0, isem1, osem0, osem1):
    cid = lax.axis_index("c")
    sid = lax.axis_index("s")
    wid = sid * NC + cid
    t0 = wid * TCW
    isems = (isem0, isem1)
    osems = (osem0, osem1)
    iota = lax.iota(jnp.int32, LANES)
    cvecs = [c16 * LANES + iota for c16 in range(C // LANES)]

    def in_copy(t, s, sem):
        gid = t0 + t
        b = gid // (TCH // B)
        rem = gid % (TCH // B)
        z = rem // (H // TRY)
        y0 = (rem % (H // TRY)) * TRY
        return pltpu.make_async_copy(
            in_hbm.at[pl.ds(b * C, C), z, pl.ds(y0, TRY), :],
            vbuf.at[s], sem)

    def out_copy(t, s, sem):
        gid = t0 + t
        b = gid // (TCH // B)
        rem = gid % (TCH // B)
        z = rem // (H // TRY)
        y0 = (rem % (H // TRY)) * TRY
        p0 = b * N + (z * H + y0) * W
        return pltpu.make_async_copy(
            tbuf.at[s], table_hbm.at[pl.ds(p0, TV)], sem)

    def transpose(s):
        def body(v4, carry):
            for u in range(4):
                v = v4 * 4 + u
                y = v // W
                x = v % W
                yv = jnp.full((LANES,), y, jnp.int32)
                xv = jnp.full((LANES,), x, jnp.int32)
                for ch in range(C // LANES):
                    vec = plsc.load_gather(vbuf.at[s], [cvecs[ch], yv, xv])
                    tbuf[s, v, pl.ds(ch * LANES, LANES)] = vec
            return carry

        lax.fori_loop(0, TV // 4, body, 0)

    in_copy(0, 0, isems[0]).start()

    @pl.when(TCW > 1)
    def _():
        in_copy(1, 1, isems[1]).start()

    def step(t, s):
        in_copy(t, s, isems[s]).wait()

        @pl.when(t >= 2)
        def _():
            out_copy(t - 2, s, osems[s]).wait()

        transpose(s)
        out_copy(t, s, osems[s]).start()

        @pl.when(t + 2 < TCW)
        def _():
            in_copy(t + 2, s, isems[s]).start()

    def pair(q, carry):
        step(2 * q, 0)
        step(2 * q + 1, 1)
        return carry

    lax.fori_loop(0, TCW // 2, pair, 0)
    out_copy(TCW - 2, 0, osems[0]).wait()
    out_copy(TCW - 1, 1, osems[1]).wait()


_tr = functools.partial(
    pl.kernel,
    out_type=jax.ShapeDtypeStruct((P, C), jnp.float32),
    mesh=plsc.VectorSubcoreMesh(core_axis_name="c", subcore_axis_name="s"),
    scratch_types=[
        pltpu.VMEM((2, C, TRY, W), jnp.float32),
        pltpu.VMEM((2, TV, C), jnp.float32),
        pltpu.SemaphoreType.DMA,
        pltpu.SemaphoreType.DMA,
        pltpu.SemaphoreType.DMA,
        pltpu.SemaphoreType.DMA,
    ],
    compiler_params=pltpu.CompilerParams(use_tc_tiling_on_sc=False, needs_layout_passes=False),
)(_tr_body)


def _sc_body(table, idx_hbm, frac_hbm, out_hbm,
             idx_v, idx1_v, frac_v, gbuf, obuf,
             isem0, isem1, gsem0, gsem1, osem0, osem1):
    cid = lax.axis_index("c")
    sid = lax.axis_index("s")
    wid = sid * NC + cid
    r0 = wid * RW
    isems = (isem0, isem1)
    gsems = (gsem0, gsem1)
    osems = (osem0, osem1)
    iota = lax.iota(jnp.int32, LANES)
    cidx = [iota + h * LANES for h in range(C // LANES)]

    def in_copies(r, s, sem):
        return (pltpu.make_async_copy(idx_hbm.at[:, r], idx_v.at[s], sem),
                pltpu.make_async_copy(frac_hbm.at[:, r], frac_v.at[s], sem))

    def gather_copies(s, sem):
        cps = []
        for k in range(4):
            cps.append(pltpu.make_async_copy(
                table.at[idx_v.at[s, k]], gbuf.at[s, 2 * k], sem))
            cps.append(pltpu.make_async_copy(
                table.at[idx1_v.at[s, k]], gbuf.at[s, 2 * k + 1], sem))
        return cps

    def out_copy(r, s, sem):
        b = r // (D * H)
        zy = r % (D * H)
        return pltpu.make_async_copy(
            obuf.at[s], out_hbm.at[pl.ds(b * C, C), zy], sem)

    def compute_idx1(s):
        for k in range(4):
            for t in range(K // LANES):
                sl = pl.ds(t * LANES, LANES)
                idx1_v[s, k, sl] = idx_v[s, k, sl] + 1

    def combine(s):
        def group(g2, carry):
            gxv = frac_v[s, 0, pl.ds(g2 * LANES, LANES)]
            gyv = frac_v[s, 1, pl.ds(g2 * LANES, LANES)]
            gzv = frac_v[s, 2, pl.ds(g2 * LANES, LANES)]
            for m in range(LANES):
                p = g2 * LANES + m
                pvec = jnp.full((LANES,), p, jnp.int32)
                tx = _lane(gxv, m)
                ty = _lane(gyv, m)
                tz = _lane(gzv, m)
                for h in range(C // LANES):
                    sl = pl.ds(h * LANES, LANES)
                    a = gbuf[s, 0, p, sl]
                    bb = gbuf[s, 1, p, sl]
                    x00 = a + tx * (bb - a)
                    a = gbuf[s, 2, p, sl]
                    bb = gbuf[s, 3, p, sl]
                    x01 = a + tx * (bb - a)
                    a = gbuf[s, 4, p, sl]
                    bb = gbuf[s, 5, p, sl]
                    x10 = a + tx * (bb - a)
                    a = gbuf[s, 6, p, sl]
                    bb = gbuf[s, 7, p, sl]
                    x11 = a + tx * (bb - a)
                    y0 = x00 + ty * (x01 - x00)
                    y1 = x10 + ty * (x11 - x10)
                    plsc.store_scatter(obuf.at[s], [cidx[h], pvec],
                                       y0 + tz * (y1 - y0))
            return carry

        lax.fori_loop(0, K // LANES, group, 0)

    for cp in in_copies(r0, 0, isems[0]):
        cp.start()
        cp.wait()
    compute_idx1(0)
    for cp in gather_copies(0, gsems[0]):
        cp.start()

    @pl.when(RW > 1)
    def _():
        for cp in in_copies(r0 + 1, 1, isems[1]):
            cp.start()

    def step(g, s):
        r = r0 + g

        @pl.when(g + 1 < RW)
        def _():
            for cp in in_copies(r + 1, 1 - s, isems[1 - s]):
                cp.wait()
            compute_idx1(1 - s)
            for cp in gather_copies(1 - s, gsems[1 - s]):
                cp.start()

        for cp in gather_copies(s, gsems[s]):
            cp.wait()

        @pl.when(g >= 2)
        def _():
            out_copy(r - 2, s, osems[s]).wait()

        combine(s)
        out_copy(r, s, osems[s]).start()

        @pl.when(g + 2 < RW)
        def _():
            for cp in in_copies(r + 2, s, isems[s]):
                cp.start()

    def pair(t, carry):
        step(2 * t, 0)
        step(2 * t + 1, 1)
        return carry

    lax.fori_loop(0, RW // 2, pair, 0)
    out_copy(r0 + RW - 2, 0, osems[0]).wait()
    out_copy(r0 + RW - 1, 1, osems[1]).wait()


_sc_warp = functools.partial(
    pl.kernel,
    out_type=jax.ShapeDtypeStruct((B * C, D * H, W), jnp.float32),
    mesh=plsc.VectorSubcoreMesh(core_axis_name="c", subcore_axis_name="s"),
    scratch_types=[
        pltpu.VMEM((2, 4, K), jnp.int32),
        pltpu.VMEM((2, 4, K), jnp.int32),
        pltpu.VMEM((2, 3, K), jnp.float32),
        pltpu.VMEM((2, 8, K, C), jnp.float32),
        pltpu.VMEM((2, C, K), jnp.float32),
        pltpu.SemaphoreType.DMA,
        pltpu.SemaphoreType.DMA,
        pltpu.SemaphoreType.DMA,
        pltpu.SemaphoreType.DMA,
        pltpu.SemaphoreType.DMA,
        pltpu.SemaphoreType.DMA,
    ],
    compiler_params=pltpu.CompilerParams(use_tc_tiling_on_sc=False, needs_layout_passes=False),
)(_sc_body)


def kernel(input, flow):
    assert input.shape == (B, C, D, H, W) and flow.shape == (B, 3, D, H, W)
    table = _tr(input.reshape(B * C, D, H, W))
    idx, frac = _prep(flow)
    out = _sc_warp(table, idx, frac)
    return out.reshape(B, C, D, H, W)


# speedup vs baseline: 1.5553x; 1.5553x over previous
import functools

import jax
import jax.numpy as jnp
from jax import lax
from jax.experimental import pallas as pl
from jax.experimental.pallas import tpu as pltpu
from jax.experimental.pallas import tpu_sc as plsc

B, C, D, H, W = 2, 32, 16, 128, 128
N = D * H * W
P = B * N
K = 128
RB = N // K
NC, NS = 2, 16
NW = NC * NS
RWB = RB // NW
LANES = 16
OROW = K * C // 128


def _prep_body(flow_ref, idx_ref, frac_ref):
    z = pl.program_id(0)
    fx = flow_ref[0, 0]
    fy = flow_ref[1, 0]
    fz = flow_ref[2, 0]
    xs = lax.broadcasted_iota(jnp.int32, (H, W), 1).astype(jnp.float32) + fx
    ys = lax.broadcasted_iota(jnp.int32, (H, W), 0).astype(jnp.float32) + fy
    zs = z.astype(jnp.float32) + fz
    bx = jnp.clip(jnp.floor(xs), 0.0, float(W - 2))
    by = jnp.clip(jnp.floor(ys), 0.0, float(H - 2))
    bz = jnp.clip(jnp.floor(zs), 0.0, float(D - 2))
    gx = jnp.clip(xs - bx, 0.0, 1.0)
    gy = jnp.clip(ys - by, 0.0, 1.0)
    gz = jnp.clip(zs - bz, 0.0, 1.0)
    base = ((bz.astype(jnp.int32) * H + by.astype(jnp.int32)) * W
            + bx.astype(jnp.int32))
    idx_ref[:, 0 * W:1 * W] = base
    idx_ref[:, 1 * W:2 * W] = base + W
    idx_ref[:, 2 * W:3 * W] = base + H * W
    idx_ref[:, 3 * W:4 * W] = base + H * W + W
    frac_ref[:, 0 * W:1 * W] = gx
    frac_ref[:, 1 * W:2 * W] = gy
    frac_ref[:, 2 * W:3 * W] = gz


_prep = pl.pallas_call(
    _prep_body,
    grid=(D,),
    in_specs=[pl.BlockSpec((3, 1, H, W), lambda i: (0, i, 0, 0))],
    out_specs=[pl.BlockSpec((H, 4 * W), lambda i: (i, 0)),
               pl.BlockSpec((H, 3 * W), lambda i: (i, 0))],
    out_shape=[jax.ShapeDtypeStruct((RB, 4 * W), jnp.int32),
               jax.ShapeDtypeStruct((RB, 3 * W), jnp.float32)],
)


def _lane(v, m):
    return lax.squeeze(lax.slice_in_dim(v, m, m + 1), (0,))


def _sc_body(table, idx_hbm, frac_hbm, out_hbm,
             idx_v, idx1_v, frac_v, gbuf, obuf,
             isem0, isem1, gsem0, gsem1, osem0, osem1):
    cid = lax.axis_index("c")
    sid = lax.axis_index("s")
    wid = sid * NC + cid
    r0 = wid * RWB
    isems = (isem0, isem1)
    gsems = (gsem0, gsem1)
    osems = (osem0, osem1)

    def in_copies(r, s, sem):
        return (pltpu.make_async_copy(idx_hbm.at[pl.ds(r * 4, 4)],
                                      idx_v.at[s], sem),
                pltpu.make_async_copy(frac_hbm.at[pl.ds(r * 3, 3)],
                                     frac_v.at[s], sem))

    def gather_copies(s, sem):
        cps = []
        for k in range(4):
            cps.append(pltpu.make_async_copy(
                table.at[idx_v.at[s, k]], gbuf.at[s, 2 * k], sem))
            cps.append(pltpu.make_async_copy(
                table.at[idx1_v.at[s, k]], gbuf.at[s, 2 * k + 1], sem))
        return cps

    def out_copy(r, s, sem):
        return pltpu.make_async_copy(
            obuf.at[s], out_hbm.at[pl.ds(r * OROW, OROW)], sem)

    def compute_idx1(s):
        for k in range(4):
            for t in range(K // LANES):
                sl = pl.ds(t * LANES, LANES)
                idx1_v[s, k, sl] = idx_v[s, k, sl] + 1

    def combine(s):
        def group(g2, carry):
            gxv = frac_v[s, 0, pl.ds(g2 * LANES, LANES)]
            gyv = frac_v[s, 1, pl.ds(g2 * LANES, LANES)]
            gzv = frac_v[s, 2, pl.ds(g2 * LANES, LANES)]
            for m in range(LANES):
                p = g2 * LANES + m
                prow = g2 * (LANES * C // 128) + (m * C) // 128
                pcol = (m * C) % 128
                tx = _lane(gxv, m)
                ty = _lane(gyv, m)
                tz = _lane(gzv, m)
                for h in range(C // LANES):
                    sl = pl.ds(h * LANES, LANES)
                    a = gbuf[s, 0, p, sl]
                    bb = gbuf[s, 1, p, sl]
                    x00 = a + tx * (bb - a)
                    a = gbuf[s, 2, p, sl]
                    bb = gbuf[s, 3, p, sl]
                    x01 = a + tx * (bb - a)
                    a = gbuf[s, 4, p, sl]
                    bb = gbuf[s, 5, p, sl]
                    x10 = a + tx * (bb - a)
                    a = gbuf[s, 6, p, sl]
                    bb = gbuf[s, 7, p, sl]
                    x11 = a + tx * (bb - a)
                    y0 = x00 + ty * (x01 - x00)
                    y1 = x10 + ty * (x11 - x10)
                    obuf[s, prow, pl.ds(pcol + h * LANES, LANES)] = \
                        y0 + tz * (y1 - y0)
            return carry

        lax.fori_loop(0, K // LANES, group, 0)

    for cp in in_copies(r0, 0, isems[0]):
        cp.start()
        cp.wait()
    compute_idx1(0)
    for cp in gather_copies(0, gsems[0]):
        cp.start()

    @pl.when(RWB > 1)
    def _():
        for cp in in_copies(r0 + 1, 1, isems[1]):
            cp.start()

    def step(g, s):
        r = r0 + g

        @pl.when(g + 1 < RWB)
        def _():
            for cp in in_copies(r + 1, 1 - s, isems[1 - s]):
                cp.wait()
            compute_idx1(1 - s)
            for cp in gather_copies(1 - s, gsems[1 - s]):
                cp.start()

        for cp in gather_copies(s, gsems[s]):
            cp.wait()

        @pl.when(g >= 2)
        def _():
            out_copy(r - 2, s, osems[s]).wait()

        combine(s)
        out_copy(r, s, osems[s]).start()

        @pl.when(g + 2 < RWB)
        def _():
            for cp in in_copies(r + 2, s, isems[s]):
                cp.start()

    def pair(t, carry):
        step(2 * t, 0)
        step(2 * t + 1, 1)
        return carry

    lax.fori_loop(0, RWB // 2, pair, 0)
    out_copy(r0 + RWB - 2, 0, osems[0]).wait()
    out_copy(r0 + RWB - 1, 1, osems[1]).wait()


_sc_warp = functools.partial(
    pl.kernel,
    out_type=jax.ShapeDtypeStruct((N * C // 128, 128), jnp.float32),
    mesh=plsc.VectorSubcoreMesh(core_axis_name="c", subcore_axis_name="s"),
    scratch_types=[
        pltpu.VMEM((2, 4, K), jnp.int32),
        pltpu.VMEM((2, 4, K), jnp.int32),
        pltpu.VMEM((2, 3, K), jnp.float32),
        pltpu.VMEM((2, 8, K, C), jnp.float32),
        pltpu.VMEM((2, OROW, 128), jnp.float32),
        pltpu.SemaphoreType.DMA,
        pltpu.SemaphoreType.DMA,
        pltpu.SemaphoreType.DMA,
        pltpu.SemaphoreType.DMA,
        pltpu.SemaphoreType.DMA,
        pltpu.SemaphoreType.DMA,
    ],
    compiler_params=pltpu.CompilerParams(use_tc_tiling_on_sc=False),
)(_sc_body)


def kernel(input, flow):
    assert input.shape == (B, C, D, H, W) and flow.shape == (B, 3, D, H, W)
    outs = []
    for b in range(B):
        table = input[b].transpose(1, 2, 3, 0).reshape(N, C)
        idx2d, frac2d = _prep(flow[b])
        idx = idx2d.reshape(RB * 4, K)
        frac = frac2d.reshape(RB * 3, K)
        rows = _sc_warp(table, idx, frac)
        outs.append(rows.reshape(D, H, W, C).transpose(3, 0, 1, 2))
    return jnp.stack(outs)


# speedup vs baseline: 1.7353x; 1.1158x over previous
import functools

import jax
import jax.numpy as jnp
from jax import lax
from jax.experimental import pallas as pl
from jax.experimental.pallas import tpu as pltpu
from jax.experimental.pallas import tpu_sc as plsc

B, C, D, H, W = 2, 32, 16, 128, 128
N = D * H * W
P = B * N
K = 128
R = P // K
NC, NS = 2, 16
NW = NC * NS
RW = R // NW
LANES = 16
OROW = K * C // 128


def _prep_body(flow_ref, idx_ref, frac_ref):
    i = pl.program_id(0)
    b = i // D
    z = i % D
    fx = flow_ref[0, 0, 0]
    fy = flow_ref[0, 1, 0]
    fz = flow_ref[0, 2, 0]
    xs = lax.broadcasted_iota(jnp.int32, (H, W), 1).astype(jnp.float32) + fx
    ys = lax.broadcasted_iota(jnp.int32, (H, W), 0).astype(jnp.float32) + fy
    zs = z.astype(jnp.float32) + fz
    bx = jnp.clip(jnp.floor(xs), 0.0, float(W - 2))
    by = jnp.clip(jnp.floor(ys), 0.0, float(H - 2))
    bz = jnp.clip(jnp.floor(zs), 0.0, float(D - 2))
    gx = jnp.clip(xs - bx, 0.0, 1.0)
    gy = jnp.clip(ys - by, 0.0, 1.0)
    gz = jnp.clip(zs - bz, 0.0, 1.0)
    base = (b * N
            + (bz.astype(jnp.int32) * H + by.astype(jnp.int32)) * W
            + bx.astype(jnp.int32))
    idx_ref[:, 0 * W:1 * W] = base
    idx_ref[:, 1 * W:2 * W] = base + W
    idx_ref[:, 2 * W:3 * W] = base + H * W
    idx_ref[:, 3 * W:4 * W] = base + H * W + W
    frac_ref[:, 0 * W:1 * W] = gx
    frac_ref[:, 1 * W:2 * W] = gy
    frac_ref[:, 2 * W:3 * W] = gz


_prep = pl.pallas_call(
    _prep_body,
    grid=(B * D,),
    in_specs=[pl.BlockSpec((1, 3, 1, H, W), lambda i: (i // D, 0, i % D, 0, 0))],
    out_specs=[pl.BlockSpec((H, 4 * W), lambda i: (i, 0)),
               pl.BlockSpec((H, 3 * W), lambda i: (i, 0))],
    out_shape=[jax.ShapeDtypeStruct((R, 4 * W), jnp.int32),
               jax.ShapeDtypeStruct((R, 3 * W), jnp.float32)],
)


def _lane(v, m):
    return lax.squeeze(lax.slice_in_dim(v, m, m + 1), (0,))


def _sc_body(table, idx_hbm, frac_hbm, out_hbm,
             idx_v, idx1_v, frac_v, gbuf, obuf,
             isem0, isem1, gsem0, gsem1, osem0, osem1):
    cid = lax.axis_index("c")
    sid = lax.axis_index("s")
    wid = sid * NC + cid
    r0 = wid * RW
    isems = (isem0, isem1)
    gsems = (gsem0, gsem1)
    osems = (osem0, osem1)

    def in_copies(r, s, sem):
        return (pltpu.make_async_copy(idx_hbm.at[pl.ds(r * 4, 4)],
                                      idx_v.at[s], sem),
                pltpu.make_async_copy(frac_hbm.at[pl.ds(r * 3, 3)],
                                     frac_v.at[s], sem))

    def gather_copies(s, sem):
        cps = []
        for k in range(4):
            cps.append(pltpu.make_async_copy(
                table.at[idx_v.at[s, k]], gbuf.at[s, 2 * k], sem))
            cps.append(pltpu.make_async_copy(
                table.at[idx1_v.at[s, k]], gbuf.at[s, 2 * k + 1], sem))
        return cps

    def out_copy(r, s, sem):
        return pltpu.make_async_copy(
            obuf.at[s], out_hbm.at[pl.ds(r * OROW, OROW)], sem)

    def compute_idx1(s):
        for k in range(4):
            for t in range(K // LANES):
                sl = pl.ds(t * LANES, LANES)
                idx1_v[s, k, sl] = idx_v[s, k, sl] + 1

    def combine(s):
        def group(g2, carry):
            gxv = frac_v[s, 0, pl.ds(g2 * LANES, LANES)]
            gyv = frac_v[s, 1, pl.ds(g2 * LANES, LANES)]
            gzv = frac_v[s, 2, pl.ds(g2 * LANES, LANES)]
            for m in range(LANES):
                p = g2 * LANES + m
                prow = g2 * (LANES * C // 128) + (m * C) // 128
                pcol = (m * C) % 128
                tx = _lane(gxv, m)
                ty = _lane(gyv, m)
                tz = _lane(gzv, m)
                for h in range(C // LANES):
                    sl = pl.ds(h * LANES, LANES)
                    a = gbuf[s, 0, p, sl]
                    bb = gbuf[s, 1, p, sl]
                    x00 = a + tx * (bb - a)
                    a = gbuf[s, 2, p, sl]
                    bb = gbuf[s, 3, p, sl]
                    x01 = a + tx * (bb - a)
                    a = gbuf[s, 4, p, sl]
                    bb = gbuf[s, 5, p, sl]
                    x10 = a + tx * (bb - a)
                    a = gbuf[s, 6, p, sl]
                    bb = gbuf[s, 7, p, sl]
                    x11 = a + tx * (bb - a)
                    y0 = x00 + ty * (x01 - x00)
                    y1 = x10 + ty * (x11 - x10)
                    obuf[s, prow, pl.ds(pcol + h * LANES, LANES)] = \
                        y0 + tz * (y1 - y0)
            return carry

        lax.fori_loop(0, K // LANES, group, 0)

    for cp in in_copies(r0, 0, isems[0]):
        cp.start()
        cp.wait()
    compute_idx1(0)
    for cp in gather_copies(0, gsems[0]):
        cp.start()

    @pl.when(RW > 1)
    def _():
        for cp in in_copies(r0 + 1, 1, isems[1]):
            cp.start()

    def step(g, s):
        r = r0 + g

        @pl.when(g + 1 < RW)
        def _():
            for cp in in_copies(r + 1, 1 - s, isems[1 - s]):
                cp.wait()
            compute_idx1(1 - s)
            for cp in gather_copies(1 - s, gsems[1 - s]):
                cp.start()

        for cp in gather_copies(s, gsems[s]):
            cp.wait()

        @pl.when(g >= 2)
        def _():
            out_copy(r - 2, s, osems[s]).wait()

        combine(s)
        out_copy(r, s, osems[s]).start()

        @pl.when(g + 2 < RW)
        def _():
            for cp in in_copies(r + 2, s, isems[s]):
                cp.start()

    def pair(t, carry):
        step(2 * t, 0)
        step(2 * t + 1, 1)
        return carry

    lax.fori_loop(0, RW // 2, pair, 0)
    out_copy(r0 + RW - 2, 0, osems[0]).wait()
    out_copy(r0 + RW - 1, 1, osems[1]).wait()


_sc_warp = functools.partial(
    pl.kernel,
    out_type=jax.ShapeDtypeStruct((P * C // 128, 128), jnp.float32),
    mesh=plsc.VectorSubcoreMesh(core_axis_name="c", subcore_axis_name="s"),
    scratch_types=[
        pltpu.VMEM((2, 4, K), jnp.int32),
        pltpu.VMEM((2, 4, K), jnp.int32),
        pltpu.VMEM((2, 3, K), jnp.float32),
        pltpu.VMEM((2, 8, K, C), jnp.float32),
        pltpu.VMEM((2, OROW, 128), jnp.float32),
        pltpu.SemaphoreType.DMA,
        pltpu.SemaphoreType.DMA,
        pltpu.SemaphoreType.DMA,
        pltpu.SemaphoreType.DMA,
        pltpu.SemaphoreType.DMA,
        pltpu.SemaphoreType.DMA,
    ],
    compiler_params=pltpu.CompilerParams(use_tc_tiling_on_sc=False),
)(_sc_body)


def kernel(input, flow):
    assert input.shape == (B, C, D, H, W) and flow.shape == (B, 3, D, H, W)
    table = input.transpose(0, 2, 3, 4, 1).reshape(P, C)
    idx2d, frac2d = _prep(flow)
    idx = idx2d.reshape(R * 4, K)
    frac = frac2d.reshape(R * 3, K)
    out_rows = _sc_warp(table, idx, frac)
    return out_rows.reshape(B, D, H, W, C).transpose(0, 4, 1, 2, 3)
